# baseline (speedup 1.0000x reference)
import functools

import jax
import jax.numpy as jnp
from jax import lax
from jax.experimental import pallas as pl
from jax.experimental.pallas import tpu as pltpu
from jax.experimental.pallas import tpu_sc as plsc

N = 524288
C = 32
K = 52429
NW = 32
CHUNK = 16384
NCH = N // CHUNK
UNROLL = 8

MIN32 = -2147483648

ROUNDS_SPEC = ((21, 11, None), (10, 11, 21), (0, 10, 10))


def _make_round(shift: int, bits: int, mask_shift):
    first = mask_shift is None
    last = shift == 0
    nbuckets = 1 << bits
    hsize = nbuckets * 16
    mesh = plsc.VectorSubcoreMesh(core_axis_name="c", subcore_axis_name="s")
    if last:
        out_type = (
            jax.ShapeDtypeStruct((NW, nbuckets), jnp.int32),
            jax.ShapeDtypeStruct((NW, nbuckets), jnp.int32),
            jax.ShapeDtypeStruct((NW, nbuckets), jnp.float32),
            jax.ShapeDtypeStruct((NW, 16), jnp.float32),
        )
    else:
        out_type = (
            jax.ShapeDtypeStruct((NW, nbuckets), jnp.int32),
            jax.ShapeDtypeStruct((NW, nbuckets), jnp.int32),
        )
    scratch = [
        pltpu.VMEM((2, CHUNK), jnp.float32),
        pltpu.VMEM((hsize,), jnp.int32),
        pltpu.VMEM((nbuckets,), jnp.int32),
        pltpu.VMEM((nbuckets,), jnp.int32),
    ]
    if last:
        scratch += [
            pltpu.VMEM((hsize,), jnp.float32),
            pltpu.VMEM((nbuckets,), jnp.float32),
            pltpu.VMEM((16,), jnp.float32),
        ]
    scratch += [
        pltpu.VMEM((C,), jnp.int32),
        pltpu.SemaphoreType.DMA,
        pltpu.SemaphoreType.DMA,
    ]

    def body(*refs):
        if first:
            (xt_hbm, cnt_hbm, cum_hbm, stage, cnt_v, cntf_v, cumf_v,
             pref_v, sem0, sem1) = refs
            pref_hbm = sum_hbm = bel_hbm = sum_v = smf_v = bel_v = None
        elif last:
            (xt_hbm, pref_hbm, cnt_hbm, cum_hbm, sum_hbm, bel_hbm, stage,
             cnt_v, cntf_v, cumf_v, sum_v, smf_v, bel_v, pref_v,
             sem0, sem1) = refs
        else:
            (xt_hbm, pref_hbm, cnt_hbm, cum_hbm, stage, cnt_v, cntf_v,
             cumf_v, pref_v, sem0, sem1) = refs
            sum_hbm = bel_hbm = sum_v = smf_v = bel_v = None

        wid = lax.axis_index("s") * 2 + lax.axis_index("c")

        zi = jnp.zeros((16,), jnp.int32)
        zf = jnp.zeros((16,), jnp.float32)

        def zero_body(i, carry):
            cnt_v[pl.ds(i * 16, 16)] = zi
            if last:
                sum_v[pl.ds(i * 16, 16)] = zf
            return carry

        lax.fori_loop(0, hsize // 16, zero_body, 0)

        iota = lax.iota(jnp.int32, 16)
        ones = jnp.ones((16,), jnp.int32)

        if not first:
            pltpu.sync_copy(pref_hbm, pref_v)
            widv = lax.broadcast_in_dim(wid, (16,), ())
            pref = plsc.load_gather(pref_v, [widv])
        else:
            pref = None

        def dma(ch, buf, sem):
            return pltpu.make_async_copy(
                xt_hbm.at[wid, pl.ds(ch * CHUNK, CHUNK)],
                stage.at[buf], sem)

        dma(0, 0, sem0).start()
        sems = (sem0, sem1)
        acc = zf
        for ch in range(NCH):
            buf = ch & 1
            dma(ch, buf, sems[buf]).wait()
            if ch + 1 < NCH:
                dma(ch + 1, 1 - buf, sems[1 - buf]).start()

            def chunk_body(j, carry):
                acc_l = carry
                vs, idxs, msks = [], [], []
                for u in range(UNROLL):
                    vs.append(stage[buf, pl.ds((j * UNROLL + u) * 16, 16)])
                for v in vs:
                    y = lax.bitcast_convert_type(v, jnp.int32)
                    m = lax.shift_right_arithmetic(y, 31)
                    key = lax.bitwise_xor(
                        y, lax.bitwise_or(m, jnp.int32(MIN32)))
                    bucket = lax.shift_right_logical(key, shift)
                    if shift + bits < 32:
                        bucket = lax.bitwise_and(
                            bucket, jnp.int32(nbuckets - 1))
                    idxs.append(bucket * 16 + iota)
                    if first:
                        msks.append(None)
                    else:
                        keyhi = lax.shift_right_logical(key, mask_shift)
                        msks.append(keyhi == pref)
                        if last:
                            bel = keyhi < pref
                            acc_l = acc_l + jnp.where(bel, v, jnp.float32(0))
                for i, (idx, msk) in enumerate(zip(idxs, msks)):
                    plsc.addupdate_scatter(cnt_v, [idx], ones, mask=msk)
                    if last:
                        plsc.addupdate_scatter(sum_v, [idx], vs[i], mask=msk)
                return acc_l

            acc = lax.fori_loop(0, CHUNK // 16 // UNROLL, chunk_body, acc)

        iota16 = iota * 16

        def fold_body(i, carry):
            ccar, scar = carry
            base = iota16 + i * 256
            ci = plsc.load_gather(cnt_v, [base])
            if last:
                si = plsc.load_gather(sum_v, [base])
            for l in range(1, 16):
                ci = ci + plsc.load_gather(cnt_v, [base + l])
                if last:
                    si = si + plsc.load_gather(sum_v, [base + l])
            cntf_v[pl.ds(i * 16, 16)] = ci
            cumf_v[pl.ds(i * 16, 16)] = (
                plsc.cumsum(ci) + lax.broadcast_in_dim(ccar, (16,), ()))
            ccar = ccar + jnp.sum(ci)
            if last:
                smf_v[pl.ds(i * 16, 16)] = (
                    plsc.cumsum(si) - si + lax.broadcast_in_dim(scar, (16,), ()))
                scar = scar + jnp.sum(si)
            return (ccar, scar)

        lax.fori_loop(0, nbuckets // 16, fold_body,
                      (jnp.int32(0), jnp.float32(0)))

        pltpu.sync_copy(cntf_v, cnt_hbm.at[wid])
        pltpu.sync_copy(cumf_v, cum_hbm.at[wid])
        if last:
            bel_v[pl.ds(0, 16)] = acc
            pltpu.sync_copy(smf_v, sum_hbm.at[wid])
            pltpu.sync_copy(bel_v, bel_hbm.at[wid])

    return pl.kernel(
        body, out_type=out_type, mesh=mesh, scratch_types=scratch,
        compiler_params=pltpu.CompilerParams(
            needs_layout_passes=False, use_tc_tiling_on_sc=True))


_SC_ROUNDS = tuple((spec, _make_round(*spec)) for spec in ROUNDS_SPEC)


def kernel(input):
    xt = input.T
    k_rem = jnp.full((C,), K, jnp.int32)
    prefix = jnp.zeros((C,), jnp.int32)
    sum_below = None
    for (shift, bits, mask_shift), fn in _SC_ROUNDS:
        nbuckets = 1 << bits
        if mask_shift is None:
            cnt, cum = fn(xt)
        elif shift != 0:
            cnt, cum = fn(xt, prefix)
        else:
            cnt, cum, smex, bel = fn(xt, prefix)
        cntm = cnt.T
        cum = cum.T
        b = jnp.argmax(cum >= k_rem[None, :], axis=0).astype(jnp.int32)
        cnt_below = jnp.take_along_axis(cum - cntm, b[None, :], 0)[0]
        if shift == 0:
            in_class_below = jnp.take_along_axis(smex.T, b[None, :], 0)[0]
            sum_below = bel.sum(axis=-1) + in_class_below
        k_rem = k_rem - cnt_below
        prefix = prefix * nbuckets + b
    ybits = jnp.where(prefix < 0, prefix ^ jnp.int32(MIN32), ~prefix)
    tval = lax.bitcast_convert_type(ybits, jnp.float32)
    return -(sum_below + k_rem.astype(jnp.float32) * tval) / jnp.float32(K)


# speedup vs baseline: 1.0120x; 1.0120x over previous
import functools

import jax
import jax.numpy as jnp
from jax import lax
from jax.experimental import pallas as pl
from jax.experimental.pallas import tpu as pltpu
from jax.experimental.pallas import tpu_sc as plsc

N = 524288
C = 32
K = 52429
NW = 32
CHUNK = 16384
NCH = N // CHUNK
UNROLL = 8

MIN32 = -2147483648

ROUNDS_SPEC = ((21, 11, None), (10, 11, 21), (0, 10, 10))


def _make_round(shift: int, bits: int, mask_shift):
    first = mask_shift is None
    last = shift == 0
    nbuckets = 1 << bits
    hsize = nbuckets * 16
    mesh = plsc.VectorSubcoreMesh(core_axis_name="c", subcore_axis_name="s")
    if last:
        out_type = (
            jax.ShapeDtypeStruct((NW, nbuckets), jnp.int32),
            jax.ShapeDtypeStruct((NW, nbuckets), jnp.int32),
            jax.ShapeDtypeStruct((NW, nbuckets), jnp.float32),
            jax.ShapeDtypeStruct((NW, 16), jnp.float32),
        )
    else:
        out_type = (
            jax.ShapeDtypeStruct((NW, nbuckets), jnp.int32),
            jax.ShapeDtypeStruct((NW, nbuckets), jnp.int32),
        )
    scratch = [
        pltpu.VMEM((2, CHUNK), jnp.float32),
        pltpu.VMEM((hsize,), jnp.int32),
        pltpu.VMEM((nbuckets,), jnp.int32),
        pltpu.VMEM((nbuckets,), jnp.int32),
    ]
    if last:
        scratch += [
            pltpu.VMEM((hsize,), jnp.float32),
            pltpu.VMEM((nbuckets,), jnp.float32),
            pltpu.VMEM((16,), jnp.float32),
        ]
    scratch += [
        pltpu.VMEM((C,), jnp.int32),
        pltpu.SemaphoreType.DMA,
        pltpu.SemaphoreType.DMA,
    ]

    def body(*refs):
        if first:
            (xt_hbm, cnt_hbm, cum_hbm, stage, cnt_v, cntf_v, cumf_v,
             pref_v, sem0, sem1) = refs
            pref_hbm = sum_hbm = bel_hbm = sum_v = smf_v = bel_v = None
        elif last:
            (xt_hbm, pref_hbm, cnt_hbm, cum_hbm, sum_hbm, bel_hbm, stage,
             cnt_v, cntf_v, cumf_v, sum_v, smf_v, bel_v, pref_v,
             sem0, sem1) = refs
        else:
            (xt_hbm, pref_hbm, cnt_hbm, cum_hbm, stage, cnt_v, cntf_v,
             cumf_v, pref_v, sem0, sem1) = refs
            sum_hbm = bel_hbm = sum_v = smf_v = bel_v = None

        wid = lax.axis_index("s") * 2 + lax.axis_index("c")

        zi = jnp.zeros((16,), jnp.int32)
        zf = jnp.zeros((16,), jnp.float32)

        def zero_body(i, carry):
            cnt_v[pl.ds(i * 16, 16)] = zi
            if last:
                sum_v[pl.ds(i * 16, 16)] = zf
            return carry

        lax.fori_loop(0, hsize // 16, zero_body, 0)

        iota = lax.iota(jnp.int32, 16)
        ones = jnp.ones((16,), jnp.int32)

        if not first:
            pltpu.sync_copy(pref_hbm, pref_v)
            widv = lax.broadcast_in_dim(wid, (16,), ())
            pref = plsc.load_gather(pref_v, [widv])
        else:
            pref = None

        def dma(ch, buf, sem):
            return pltpu.make_async_copy(
                xt_hbm.at[wid, pl.ds(ch * CHUNK, CHUNK)],
                stage.at[buf], sem)

        dma(0, 0, sem0).start()
        sems = (sem0, sem1)
        acc = zf
        for ch in range(NCH):
            buf = ch & 1
            dma(ch, buf, sems[buf]).wait()
            if ch + 1 < NCH:
                dma(ch + 1, 1 - buf, sems[1 - buf]).start()

            def chunk_body(j, carry):
                acc_l = carry
                vs, idxs, msks = [], [], []
                for u in range(UNROLL):
                    vs.append(stage[buf, pl.ds((j * UNROLL + u) * 16, 16)])
                for v in vs:
                    y = lax.bitcast_convert_type(v, jnp.int32)
                    m = lax.shift_right_arithmetic(y, 31)
                    key = lax.bitwise_xor(
                        y, lax.bitwise_or(m, jnp.int32(MIN32)))
                    bucket = lax.shift_right_logical(key, shift)
                    if shift + bits < 32:
                        bucket = lax.bitwise_and(
                            bucket, jnp.int32(nbuckets - 1))
                    idxs.append(bucket * 16 + iota)
                    if first:
                        msks.append(None)
                    else:
                        keyhi = lax.shift_right_logical(key, mask_shift)
                        msks.append(keyhi == pref)
                        if last:
                            bel = keyhi < pref
                            acc_l = acc_l + jnp.where(bel, v, jnp.float32(0))
                for i, (idx, msk) in enumerate(zip(idxs, msks)):
                    plsc.addupdate_scatter(cnt_v, [idx], ones, mask=msk)
                    if last:
                        plsc.addupdate_scatter(sum_v, [idx], vs[i], mask=msk)
                return acc_l

            acc = lax.fori_loop(0, CHUNK // 16 // UNROLL, chunk_body, acc)

        iota16 = iota * 16

        def fold_body(i, carry):
            ccar, scar = carry
            base = iota16 + i * 256
            ci = plsc.load_gather(cnt_v, [base])
            if last:
                si = plsc.load_gather(sum_v, [base])
            for l in range(1, 16):
                ci = ci + plsc.load_gather(cnt_v, [base + l])
                if last:
                    si = si + plsc.load_gather(sum_v, [base + l])
            cntf_v[pl.ds(i * 16, 16)] = ci
            cumf_v[pl.ds(i * 16, 16)] = (
                plsc.cumsum(ci) + lax.broadcast_in_dim(ccar, (16,), ()))
            ccar = ccar + jnp.sum(ci)
            if last:
                smf_v[pl.ds(i * 16, 16)] = (
                    plsc.cumsum(si) - si + lax.broadcast_in_dim(scar, (16,), ()))
                scar = scar + jnp.sum(si)
            return (ccar, scar)

        lax.fori_loop(0, nbuckets // 16, fold_body,
                      (jnp.int32(0), jnp.float32(0)))

        pltpu.sync_copy(cntf_v, cnt_hbm.at[wid])
        pltpu.sync_copy(cumf_v, cum_hbm.at[wid])
        if last:
            bel_v[pl.ds(0, 16)] = acc
            pltpu.sync_copy(smf_v, sum_hbm.at[wid])
            pltpu.sync_copy(bel_v, bel_hbm.at[wid])

    return pl.kernel(
        body, out_type=out_type, mesh=mesh, scratch_types=scratch,
        compiler_params=pltpu.CompilerParams(
            needs_layout_passes=False, use_tc_tiling_on_sc=False))


_SC_ROUNDS = tuple((spec, _make_round(*spec)) for spec in ROUNDS_SPEC)


def kernel(input):
    xt = input.T
    k_rem = jnp.full((C,), K, jnp.int32)
    prefix = jnp.zeros((C,), jnp.int32)
    sum_below = None
    for (shift, bits, mask_shift), fn in _SC_ROUNDS:
        nbuckets = 1 << bits
        if mask_shift is None:
            cnt, cum = fn(xt)
        elif shift != 0:
            cnt, cum = fn(xt, prefix)
        else:
            cnt, cum, smex, bel = fn(xt, prefix)
        cntm = cnt.T
        cum = cum.T
        b = jnp.argmax(cum >= k_rem[None, :], axis=0).astype(jnp.int32)
        cnt_below = jnp.take_along_axis(cum - cntm, b[None, :], 0)[0]
        if shift == 0:
            in_class_below = jnp.take_along_axis(smex.T, b[None, :], 0)[0]
            sum_below = bel.sum(axis=-1) + in_class_below
        k_rem = k_rem - cnt_below
        prefix = prefix * nbuckets + b
    ybits = jnp.where(prefix < 0, prefix ^ jnp.int32(MIN32), ~prefix)
    tval = lax.bitcast_convert_type(ybits, jnp.float32)
    return -(sum_below + k_rem.astype(jnp.float32) * tval) / jnp.float32(K)
